# baseline (speedup 1.0000x reference)
import functools

import numpy as np
import jax
import jax.numpy as jnp
from jax import lax
from jax.experimental import pallas as pl
from jax.experimental.pallas import tpu as pltpu
from jax.experimental.pallas import tpu_sc as plsc

_F32 = jnp.float32
_BF16 = jnp.bfloat16
_I32 = jnp.int32
_BK = 128


def _gelu(x):
    return 0.5 * x * (1.0 + lax.erf(x * np.float32(0.7071067811865476)))


def _pe_const(T, D):
    pos = np.arange(T)[:, None].astype(np.float32)
    div = np.exp(np.arange(0, D, 2).astype(np.float32) * (-np.log(10000.0) / D))
    pe = np.zeros((T, D), dtype=np.float32)
    pe[:, 0::2] = np.sin(pos * div)
    pe[:, 1::2] = np.cos(pos * div)
    return jnp.asarray(pe)


def _router(x, pe, rW1, rb1, rW2):
    T, D = x.shape
    H = rW1.shape[1]
    E = rW2.shape[1]
    TB, HB = 512, 768
    nT, nH = T // TB, H // HB

    def body(x_ref, pe_ref, w1_ref, b1_ref, w2_ref, h_ref, lg_ref):
        j = pl.program_id(1)
        h = x_ref[...] + pe_ref[...]

        @pl.when(j == 0)
        def _():
            h_ref[...] = h

        t = _gelu(jnp.dot(h, w1_ref[...], preferred_element_type=_F32) + b1_ref[...])
        contrib = jnp.dot(t, w2_ref[...], preferred_element_type=_F32)

        @pl.when(j == 0)
        def _():
            lg_ref[...] = contrib

        @pl.when(j != 0)
        def _():
            lg_ref[...] += contrib

    return pl.pallas_call(
        body,
        grid=(nT, nH),
        in_specs=[
            pl.BlockSpec((TB, D), lambda i, j: (i, 0)),
            pl.BlockSpec((TB, D), lambda i, j: (i, 0)),
            pl.BlockSpec((D, HB), lambda i, j: (0, j)),
            pl.BlockSpec((1, HB), lambda i, j: (0, j)),
            pl.BlockSpec((HB, E), lambda i, j: (j, 0)),
        ],
        out_specs=[
            pl.BlockSpec((TB, D), lambda i, j: (i, 0)),
            pl.BlockSpec((TB, E), lambda i, j: (i, 0)),
        ],
        out_shape=[
            jax.ShapeDtypeStruct((T, D), _F32),
            jax.ShapeDtypeStruct((T, E), _F32),
        ],
    )(x, pe, rW1, rb1.reshape(1, H), rW2)


def _route(raw, rb2, temp, NB):
    T, E = raw.shape

    def body(raw_ref, rb2_ref, temp_ref, lg_ref, topv_ref, d0_ref, d1_ref,
             bexp_ref, loss_ref):
        inv = 1.0 / (temp_ref[0, 0] + np.float32(1e-6))
        lg = (raw_ref[...] + rb2_ref[...]) * inv
        lg_ref[...] = lg
        m = jnp.max(lg, axis=-1, keepdims=True)
        ex = jnp.exp(lg - m)
        probs = ex / jnp.sum(ex, axis=-1, keepdims=True)
        iota8 = lax.broadcasted_iota(_I32, (T, E), 1)
        m1 = jnp.max(probs, axis=-1, keepdims=True)
        i1 = jnp.min(jnp.where(probs == m1, iota8, E), axis=-1, keepdims=True)
        A0 = iota8 == i1
        pr2 = jnp.where(A0, np.float32(-1.0), probs)
        m2 = jnp.max(pr2, axis=-1, keepdims=True)
        i2 = jnp.min(jnp.where(pr2 == m2, iota8, E), axis=-1, keepdims=True)
        A1 = iota8 == i2
        topv_ref[...] = jnp.concatenate([m1, m2], axis=-1)

        A0f = A0.astype(_F32)
        A1f = A1.astype(_F32)
        c0 = jnp.sum(A0f, axis=0, keepdims=True)
        c1 = jnp.sum(A1f, axis=0, keepdims=True)

        el = jnp.sum(probs, axis=0, keepdims=True)
        loss = jnp.mean(el * el) * np.float32(E)
        loss_ref[...] = loss[None, None]

        cnt = (c0 + c1).astype(_I32)
        nb_e = (cnt + (_BK - 1)) // _BK
        inc = nb_e
        sh = 1
        while sh < E:
            inc = inc + jnp.concatenate(
                [jnp.zeros((1, sh), _I32), inc[:, :-sh]], axis=-1)
            sh *= 2
        bstart = inc - nb_e
        slot_base = (bstart * _BK).astype(_F32)
        bid = lax.broadcasted_iota(_I32, (NB, 1), 0)
        bexp = jnp.sum((bid >= inc).astype(_I32), axis=-1, keepdims=True)
        used = bid < jnp.max(inc, axis=-1, keepdims=True)
        bexp_ref[...] = jnp.where(used, jnp.minimum(bexp, E - 1), E)

        def excl0(Af):
            c = Af
            s = 1
            while s < T:
                c = c + jnp.concatenate(
                    [jnp.zeros((s, E), _F32), c[:-s]], axis=0)
                s *= 2
            return c - Af

        R0 = excl0(A0f)
        R1 = excl0(A1f)
        d0 = jnp.sum(A0f * (slot_base + R0), axis=-1, keepdims=True)
        d1 = jnp.sum(A1f * (slot_base + c0 + R1), axis=-1, keepdims=True)
        d0_ref[...] = d0.astype(_I32)
        d1_ref[...] = d1.astype(_I32)

    return pl.pallas_call(
        body,
        out_shape=[
            jax.ShapeDtypeStruct((T, E), _F32),
            jax.ShapeDtypeStruct((T, 2), _F32),
            jax.ShapeDtypeStruct((T, 1), _I32),
            jax.ShapeDtypeStruct((T, 1), _I32),
            jax.ShapeDtypeStruct((NB, 1), _I32),
            jax.ShapeDtypeStruct((1, 1), _F32),
        ],
    )(raw, rb2.reshape(1, E), temp.reshape(1, 1))


def _dispatch_sc(h, dest_all, P):
    T, D = h.shape
    NPAIR = dest_all.shape[0]
    info = plsc.get_sparse_core_info()
    NW = info.num_cores * info.num_subcores
    per_w = NPAIR // NW
    mesh = plsc.VectorSubcoreMesh(core_axis_name="c", subcore_axis_name="s")

    @functools.partial(
        pl.kernel,
        mesh=mesh,
        out_type=jax.ShapeDtypeStruct((P, D), _F32),
        scratch_types=[
            pltpu.VMEM((per_w,), _I32),
            pltpu.VMEM((per_w, D), _F32),
            pltpu.SemaphoreType.DMA,
        ],
    )
    def k(h_hbm, dest_hbm, xg_hbm, idx_v, rows_v, sem):
        wid = lax.axis_index("s") * info.num_cores + lax.axis_index("c")
        p0 = wid * per_w
        t0 = lax.rem(p0, T)
        pltpu.sync_copy(dest_hbm.at[pl.ds(p0, per_w)], idx_v)
        pltpu.sync_copy(h_hbm.at[pl.ds(t0, per_w)], rows_v)
        pltpu.async_copy(rows_v, xg_hbm.at[idx_v], sem).wait()

    return k(h, dest_all)


def _combine_sc(eo_pad, dest_all):
    P, D = eo_pad.shape
    NPAIR = dest_all.shape[0]
    info = plsc.get_sparse_core_info()
    NW = info.num_cores * info.num_subcores
    per_w = NPAIR // NW
    mesh = plsc.VectorSubcoreMesh(core_axis_name="c", subcore_axis_name="s")

    @functools.partial(
        pl.kernel,
        mesh=mesh,
        out_type=jax.ShapeDtypeStruct((NPAIR, D), _F32),
        scratch_types=[
            pltpu.VMEM((per_w,), _I32),
            pltpu.VMEM((per_w, D), _F32),
            pltpu.SemaphoreType.DMA,
        ],
    )
    def k(eo_hbm, dest_hbm, g_hbm, idx_v, rows_v, sem):
        wid = lax.axis_index("s") * info.num_cores + lax.axis_index("c")
        p0 = wid * per_w
        pltpu.sync_copy(dest_hbm.at[pl.ds(p0, per_w)], idx_v)
        pltpu.async_copy(eo_hbm.at[idx_v], rows_v, sem).wait()
        pltpu.sync_copy(rows_v, g_hbm.at[pl.ds(p0, per_w)])

    return k(eo_pad, dest_all)


def _experts(xg, bexp, eWg, ebg, eWa, eba, eWb, ebb, eg, eb):
    P, D = xg.shape
    E, SUB = ebg.shape
    NB = P // _BK
    DS = SUB * D
    wa_cat = eWa.transpose(0, 2, 1, 3).reshape(E, D, DS)
    ba_cat = eba.reshape(E, 1, DS)
    wb_cat = eWb.reshape(E, DS, D)

    def body(bexp_ref, xg_ref, wg_ref, bg_ref, wa_ref, ba_ref, wb_ref,
             bb_ref, g_ref, b_ref, out_ref):
        b = pl.program_id(0)

        @pl.when(bexp_ref[b] < E)
        def _():
            xgb = xg_ref[...]
            gate_l = (jnp.dot(xgb, wg_ref[0], preferred_element_type=_F32)
                      + bg_ref[0])
            gm = jnp.max(gate_l, axis=-1, keepdims=True)
            ge = jnp.exp(gate_l - gm)
            gate = ge / jnp.sum(ge, axis=-1, keepdims=True)
            a = _gelu(jnp.dot(xgb, wa_ref[0], preferred_element_type=_F32)
                      + ba_ref[0])
            ag = jnp.concatenate(
                [a[:, s * D:(s + 1) * D] * gate[:, s:s + 1]
                 for s in range(SUB)], axis=1)
            mix = (jnp.dot(ag, wb_ref[0], preferred_element_type=_F32)
                   + jnp.dot(gate, bb_ref[0], preferred_element_type=_F32))
            r = xgb + mix
            mu = jnp.mean(r, axis=-1, keepdims=True)
            var = jnp.mean((r - mu) ** 2, axis=-1, keepdims=True)
            eo = (r - mu) * lax.rsqrt(var + np.float32(1e-5))
            out_ref[...] = eo * g_ref[0] + b_ref[0]

    def _emap(b, m):
        return (jnp.minimum(m[b], E - 1), 0, 0)

    grid_spec = pltpu.PrefetchScalarGridSpec(
        num_scalar_prefetch=1,
        grid=(NB,),
        in_specs=[
            pl.BlockSpec((_BK, D), lambda b, m: (b, 0)),
            pl.BlockSpec((1, D, SUB), _emap),
            pl.BlockSpec((1, 1, SUB), _emap),
            pl.BlockSpec((1, D, DS), _emap),
            pl.BlockSpec((1, 1, DS), _emap),
            pl.BlockSpec((1, DS, D), _emap),
            pl.BlockSpec((1, SUB, D), _emap),
            pl.BlockSpec((1, 1, D), _emap),
            pl.BlockSpec((1, 1, D), _emap),
        ],
        out_specs=pl.BlockSpec((_BK, D), lambda b, m: (b, 0)),
    )
    return pl.pallas_call(
        body,
        grid_spec=grid_spec,
        out_shape=jax.ShapeDtypeStruct((P, D), _F32),
    )(bexp, xg, eWg, ebg.reshape(E, 1, SUB), wa_cat, ba_cat, wb_cat, ebb,
      eg.reshape(E, 1, D), eb.reshape(E, 1, D))


def _final(g0, g1, topv, cW1, cb1, cW2, cb2, cg, cb):
    T, D = g0.shape
    D2 = cW1.shape[1]
    TB = 256
    nT = T // TB

    def body(g0_ref, g1_ref, tv_ref, w1_ref, b1_ref, w2_ref, b2_ref,
             g_ref, b_ref, out_ref):
        tv = tv_ref[...]
        comb = tv[:, 0:1] * g0_ref[...] + tv[:, 1:2] * g1_ref[...]
        y = _gelu(jnp.dot(comb, w1_ref[...], preferred_element_type=_F32)
                  + b1_ref[...])
        z = jnp.dot(y, w2_ref[...], preferred_element_type=_F32) + b2_ref[...]
        mu = jnp.mean(z, axis=-1, keepdims=True)
        var = jnp.mean((z - mu) ** 2, axis=-1, keepdims=True)
        out_ref[...] = ((z - mu) * lax.rsqrt(var + np.float32(1e-5))
                        * g_ref[...] + b_ref[...])

    return pl.pallas_call(
        body,
        grid=(nT,),
        in_specs=[
            pl.BlockSpec((TB, D), lambda i: (i, 0)),
            pl.BlockSpec((TB, D), lambda i: (i, 0)),
            pl.BlockSpec((TB, 2), lambda i: (i, 0)),
            pl.BlockSpec((D, D2), lambda i: (0, 0)),
            pl.BlockSpec((1, D2), lambda i: (0, 0)),
            pl.BlockSpec((D2, D), lambda i: (0, 0)),
            pl.BlockSpec((1, D), lambda i: (0, 0)),
            pl.BlockSpec((1, D), lambda i: (0, 0)),
            pl.BlockSpec((1, D), lambda i: (0, 0)),
        ],
        out_specs=pl.BlockSpec((TB, D), lambda i: (i, 0)),
        out_shape=jax.ShapeDtypeStruct((T, D), _F32),
    )(g0, g1, topv, cW1, cb1.reshape(1, D2), cW2, cb2.reshape(1, D),
      cg.reshape(1, D), cb.reshape(1, D))


def kernel(x, rW1, rb1, rW2, rb2, temp, eWg, ebg, eWa, eba, eWb, ebb,
           eg, eb, cW1, cb1, cW2, cb2, cg, cb):
    Bsz, T, D = x.shape
    E = rW2.shape[1]
    K = 2
    NPAIR = Bsz * T * K
    NB = -(-(NPAIR + E * (_BK - 1)) // _BK)
    P = NB * _BK

    xs = x.reshape(Bsz * T, D)
    pe = _pe_const(T, D)
    if Bsz > 1:
        pe = jnp.tile(pe, (Bsz, 1))

    h, raw = _router(xs, pe, rW1, rb1, rW2)
    logits, topv, d0, d1, bexp, loss = _route(raw, rb2, temp, NB)
    dest_all = jnp.concatenate([d0.reshape(-1), d1.reshape(-1)])
    xg = _dispatch_sc(h, dest_all, P)
    eo_pad = _experts(xg, bexp.reshape(NB), eWg, ebg, eWa, eba, eWb, ebb,
                      eg, eb)
    g_all = _combine_sc(eo_pad, dest_all)
    out = _final(g_all[:Bsz * T], g_all[Bsz * T:], topv, cW1, cb1, cW2,
                 cb2, cg, cb)
    return (out.reshape(Bsz, T, D), logits.reshape(Bsz, T, E),
            loss.reshape(()))


# speedup vs baseline: 1.1464x; 1.1464x over previous
import functools

import numpy as np
import jax
import jax.numpy as jnp
from jax import lax
from jax.experimental import pallas as pl
from jax.experimental.pallas import tpu as pltpu
from jax.experimental.pallas import tpu_sc as plsc

_F32 = jnp.float32
_BF16 = jnp.bfloat16
_I32 = jnp.int32
_BK = 128


def _gelu(x):
    return 0.5 * x * (1.0 + lax.erf(x * np.float32(0.7071067811865476)))


def _pe_const(T, D):
    pos = np.arange(T)[:, None].astype(np.float32)
    div = np.exp(np.arange(0, D, 2).astype(np.float32) * (-np.log(10000.0) / D))
    pe = np.zeros((T, D), dtype=np.float32)
    pe[:, 0::2] = np.sin(pos * div)
    pe[:, 1::2] = np.cos(pos * div)
    return jnp.asarray(pe)


def _router(x, pe, rW1, rb1, rW2):
    T, D = x.shape
    H = rW1.shape[1]
    E = rW2.shape[1]
    TB, HB = 512, 768
    nT, nH = T // TB, H // HB

    def body(x_ref, pe_ref, w1_ref, b1_ref, w2_ref, h_ref, lg_ref):
        j = pl.program_id(1)
        h = x_ref[...] + pe_ref[...]

        @pl.when(j == 0)
        def _():
            h_ref[...] = h

        t = _gelu(jnp.dot(h, w1_ref[...], preferred_element_type=_F32) + b1_ref[...])
        contrib = jnp.dot(t, w2_ref[...], preferred_element_type=_F32)

        @pl.when(j == 0)
        def _():
            lg_ref[...] = contrib

        @pl.when(j != 0)
        def _():
            lg_ref[...] += contrib

    return pl.pallas_call(
        body,
        grid=(nT, nH),
        in_specs=[
            pl.BlockSpec((TB, D), lambda i, j: (i, 0)),
            pl.BlockSpec((TB, D), lambda i, j: (i, 0)),
            pl.BlockSpec((D, HB), lambda i, j: (0, j)),
            pl.BlockSpec((1, HB), lambda i, j: (0, j)),
            pl.BlockSpec((HB, E), lambda i, j: (j, 0)),
        ],
        out_specs=[
            pl.BlockSpec((TB, D), lambda i, j: (i, 0)),
            pl.BlockSpec((TB, E), lambda i, j: (i, 0)),
        ],
        out_shape=[
            jax.ShapeDtypeStruct((T, D), _F32),
            jax.ShapeDtypeStruct((T, E), _F32),
        ],
    )(x, pe, rW1, rb1.reshape(1, H), rW2)


def _route(raw, rb2, temp, NB):
    T, E = raw.shape

    def body(raw_ref, rb2_ref, temp_ref, lg_ref, topv_ref, d0_ref, d1_ref,
             bexp_ref, loss_ref):
        inv = 1.0 / (temp_ref[0, 0] + np.float32(1e-6))
        lg = (raw_ref[...] + rb2_ref[...]) * inv
        lg_ref[...] = lg
        m = jnp.max(lg, axis=-1, keepdims=True)
        ex = jnp.exp(lg - m)
        probs = ex / jnp.sum(ex, axis=-1, keepdims=True)
        iota8 = lax.broadcasted_iota(_I32, (T, E), 1)
        m1 = jnp.max(probs, axis=-1, keepdims=True)
        i1 = jnp.min(jnp.where(probs == m1, iota8, E), axis=-1, keepdims=True)
        A0 = iota8 == i1
        pr2 = jnp.where(A0, np.float32(-1.0), probs)
        m2 = jnp.max(pr2, axis=-1, keepdims=True)
        i2 = jnp.min(jnp.where(pr2 == m2, iota8, E), axis=-1, keepdims=True)
        A1 = iota8 == i2
        topv_ref[...] = jnp.concatenate([m1, m2], axis=-1)

        A0f = A0.astype(_F32)
        A1f = A1.astype(_F32)
        c0 = jnp.sum(A0f, axis=0, keepdims=True)
        c1 = jnp.sum(A1f, axis=0, keepdims=True)

        el = jnp.sum(probs, axis=0, keepdims=True)
        loss = jnp.mean(el * el) * np.float32(E)
        loss_ref[...] = loss[None, None]

        cnt = (c0 + c1).astype(_I32)
        nb_e = (cnt + (_BK - 1)) // _BK
        inc = nb_e
        sh = 1
        while sh < E:
            inc = inc + jnp.concatenate(
                [jnp.zeros((1, sh), _I32), inc[:, :-sh]], axis=-1)
            sh *= 2
        bstart = inc - nb_e
        slot_base = (bstart * _BK).astype(_F32)
        bid = lax.broadcasted_iota(_I32, (NB, 1), 0)
        bexp = jnp.sum((bid >= inc).astype(_I32), axis=-1, keepdims=True)
        used = bid < jnp.max(inc, axis=-1, keepdims=True)
        bexp_ref[...] = jnp.where(used, jnp.minimum(bexp, E - 1), E)

        def excl0(Af):
            c = Af
            s = 1
            while s < T:
                c = c + jnp.concatenate(
                    [jnp.zeros((s, E), _F32), c[:-s]], axis=0)
                s *= 2
            return c - Af

        R0 = excl0(A0f)
        R1 = excl0(A1f)
        d0 = jnp.sum(A0f * (slot_base + R0), axis=-1, keepdims=True)
        d1 = jnp.sum(A1f * (slot_base + c0 + R1), axis=-1, keepdims=True)
        d0_ref[...] = d0.astype(_I32)
        d1_ref[...] = d1.astype(_I32)

    return pl.pallas_call(
        body,
        out_shape=[
            jax.ShapeDtypeStruct((T, E), _F32),
            jax.ShapeDtypeStruct((T, 2), _F32),
            jax.ShapeDtypeStruct((T, 1), _I32),
            jax.ShapeDtypeStruct((T, 1), _I32),
            jax.ShapeDtypeStruct((NB, 1), _I32),
            jax.ShapeDtypeStruct((1, 1), _F32),
        ],
    )(raw, rb2.reshape(1, E), temp.reshape(1, 1))


def _dispatch_sc(h, dest_all, P):
    T, D = h.shape
    NPAIR = dest_all.shape[0]
    info = plsc.get_sparse_core_info()
    NW = info.num_cores * info.num_subcores
    per_w = NPAIR // NW
    mesh = plsc.VectorSubcoreMesh(core_axis_name="c", subcore_axis_name="s")

    @functools.partial(
        pl.kernel,
        mesh=mesh,
        out_type=jax.ShapeDtypeStruct((P, D), _F32),
        scratch_types=[
            pltpu.VMEM((per_w,), _I32),
            pltpu.VMEM((per_w, D), _F32),
            pltpu.SemaphoreType.DMA,
        ],
    )
    def k(h_hbm, dest_hbm, xg_hbm, idx_v, rows_v, sem):
        wid = lax.axis_index("s") * info.num_cores + lax.axis_index("c")
        p0 = wid * per_w
        t0 = lax.rem(p0, T)
        pltpu.sync_copy(dest_hbm.at[pl.ds(p0, per_w)], idx_v)
        pltpu.sync_copy(h_hbm.at[pl.ds(t0, per_w)], rows_v)
        pltpu.async_copy(rows_v, xg_hbm.at[idx_v], sem).wait()

    return k(h, dest_all)


def _combine_sc(eo_pad, dest_all):
    P, D = eo_pad.shape
    NPAIR = dest_all.shape[0]
    info = plsc.get_sparse_core_info()
    NW = info.num_cores * info.num_subcores
    per_w = NPAIR // NW
    mesh = plsc.VectorSubcoreMesh(core_axis_name="c", subcore_axis_name="s")

    @functools.partial(
        pl.kernel,
        mesh=mesh,
        out_type=jax.ShapeDtypeStruct((NPAIR, D), _F32),
        scratch_types=[
            pltpu.VMEM((per_w,), _I32),
            pltpu.VMEM((per_w, D), _F32),
            pltpu.SemaphoreType.DMA,
        ],
    )
    def k(eo_hbm, dest_hbm, g_hbm, idx_v, rows_v, sem):
        wid = lax.axis_index("s") * info.num_cores + lax.axis_index("c")
        p0 = wid * per_w
        pltpu.sync_copy(dest_hbm.at[pl.ds(p0, per_w)], idx_v)
        pltpu.async_copy(eo_hbm.at[idx_v], rows_v, sem).wait()
        pltpu.sync_copy(rows_v, g_hbm.at[pl.ds(p0, per_w)])

    return k(eo_pad, dest_all)


def _experts(xg, bexp, eWg, ebg, eWa, eba, eWb, ebb, eg, eb):
    P, D = xg.shape
    E, SUB = ebg.shape
    NB = P // _BK
    DS = SUB * D
    wb_cat = eWb.reshape(E, DS, D)

    def body(bexp_ref, xg_ref, wg_ref, bg_ref, wa_ref, ba_ref, wb_ref,
             bb_ref, g_ref, b_ref, out_ref):
        b = pl.program_id(0)

        @pl.when(bexp_ref[b] < E)
        def _():
            xgb = xg_ref[...]
            gate_l = (jnp.dot(xgb, wg_ref[0], preferred_element_type=_F32)
                      + bg_ref[0])
            gm = jnp.max(gate_l, axis=-1, keepdims=True)
            ge = jnp.exp(gate_l - gm)
            gate = ge / jnp.sum(ge, axis=-1, keepdims=True)
            ag = jnp.concatenate(
                [gate[:, s:s + 1]
                 * _gelu(jnp.dot(xgb, wa_ref[0, s],
                                 preferred_element_type=_F32)
                         + ba_ref[0, s][None, :])
                 for s in range(SUB)], axis=1)
            mix = (jnp.dot(ag, wb_ref[0], preferred_element_type=_F32)
                   + jnp.dot(gate, bb_ref[0], preferred_element_type=_F32))
            r = xgb + mix
            mu = jnp.mean(r, axis=-1, keepdims=True)
            var = jnp.mean((r - mu) ** 2, axis=-1, keepdims=True)
            eo = (r - mu) * lax.rsqrt(var + np.float32(1e-5))
            out_ref[...] = eo * g_ref[0] + b_ref[0]

    def _emap(b, m):
        return (jnp.minimum(m[b], E - 1), 0, 0)

    grid_spec = pltpu.PrefetchScalarGridSpec(
        num_scalar_prefetch=1,
        grid=(NB,),
        in_specs=[
            pl.BlockSpec((_BK, D), lambda b, m: (b, 0)),
            pl.BlockSpec((1, D, SUB), _emap),
            pl.BlockSpec((1, 1, SUB), _emap),
            pl.BlockSpec((1, SUB, D, D),
                         lambda b, m: (jnp.minimum(m[b], E - 1), 0, 0, 0)),
            pl.BlockSpec((1, SUB, D), _emap),
            pl.BlockSpec((1, DS, D), _emap),
            pl.BlockSpec((1, SUB, D), _emap),
            pl.BlockSpec((1, 1, D), _emap),
            pl.BlockSpec((1, 1, D), _emap),
        ],
        out_specs=pl.BlockSpec((_BK, D), lambda b, m: (b, 0)),
    )
    return pl.pallas_call(
        body,
        grid_spec=grid_spec,
        out_shape=jax.ShapeDtypeStruct((P, D), _F32),
    )(bexp, xg, eWg, ebg.reshape(E, 1, SUB), eWa, eba, wb_cat, ebb,
      eg.reshape(E, 1, D), eb.reshape(E, 1, D))


def _final(g0, g1, topv, cW1, cb1, cW2, cb2, cg, cb):
    T, D = g0.shape
    D2 = cW1.shape[1]
    TB = 256
    nT = T // TB

    def body(g0_ref, g1_ref, tv_ref, w1_ref, b1_ref, w2_ref, b2_ref,
             g_ref, b_ref, out_ref):
        tv = tv_ref[...]
        comb = tv[:, 0:1] * g0_ref[...] + tv[:, 1:2] * g1_ref[...]
        y = _gelu(jnp.dot(comb, w1_ref[...], preferred_element_type=_F32)
                  + b1_ref[...])
        z = jnp.dot(y, w2_ref[...], preferred_element_type=_F32) + b2_ref[...]
        mu = jnp.mean(z, axis=-1, keepdims=True)
        var = jnp.mean((z - mu) ** 2, axis=-1, keepdims=True)
        out_ref[...] = ((z - mu) * lax.rsqrt(var + np.float32(1e-5))
                        * g_ref[...] + b_ref[...])

    return pl.pallas_call(
        body,
        grid=(nT,),
        in_specs=[
            pl.BlockSpec((TB, D), lambda i: (i, 0)),
            pl.BlockSpec((TB, D), lambda i: (i, 0)),
            pl.BlockSpec((TB, 2), lambda i: (i, 0)),
            pl.BlockSpec((D, D2), lambda i: (0, 0)),
            pl.BlockSpec((1, D2), lambda i: (0, 0)),
            pl.BlockSpec((D2, D), lambda i: (0, 0)),
            pl.BlockSpec((1, D), lambda i: (0, 0)),
            pl.BlockSpec((1, D), lambda i: (0, 0)),
            pl.BlockSpec((1, D), lambda i: (0, 0)),
        ],
        out_specs=pl.BlockSpec((TB, D), lambda i: (i, 0)),
        out_shape=jax.ShapeDtypeStruct((T, D), _F32),
    )(g0, g1, topv, cW1, cb1.reshape(1, D2), cW2, cb2.reshape(1, D),
      cg.reshape(1, D), cb.reshape(1, D))


def kernel(x, rW1, rb1, rW2, rb2, temp, eWg, ebg, eWa, eba, eWb, ebb,
           eg, eb, cW1, cb1, cW2, cb2, cg, cb):
    Bsz, T, D = x.shape
    E = rW2.shape[1]
    K = 2
    NPAIR = Bsz * T * K
    NB = -(-(NPAIR + E * (_BK - 1)) // _BK)
    P = NB * _BK

    xs = x.reshape(Bsz * T, D)
    pe = _pe_const(T, D)
    if Bsz > 1:
        pe = jnp.tile(pe, (Bsz, 1))

    h, raw = _router(xs, pe, rW1, rb1, rW2)
    logits, topv, d0, d1, bexp, loss = _route(raw, rb2, temp, NB)
    dest_all = jnp.concatenate([d0.reshape(-1), d1.reshape(-1)])
    xg = _dispatch_sc(h, dest_all, P)
    eo_pad = _experts(xg, bexp.reshape(NB), eWg, ebg, eWa, eba, eWb, ebb,
                      eg, eb)
    g_all = _combine_sc(eo_pad, dest_all)
    out = _final(g_all[:Bsz * T], g_all[Bsz * T:], topv, cW1, cb1, cW2,
                 cb2, cg, cb)
    return (out.reshape(Bsz, T, D), logits.reshape(Bsz, T, E),
            loss.reshape(()))


# speedup vs baseline: 1.1972x; 1.0444x over previous
import functools

import numpy as np
import jax
import jax.numpy as jnp
from jax import lax
from jax.experimental import pallas as pl
from jax.experimental.pallas import tpu as pltpu
from jax.experimental.pallas import tpu_sc as plsc

_F32 = jnp.float32
_BF16 = jnp.bfloat16
_I32 = jnp.int32
_BK = 128


def _gelu(x):
    return 0.5 * x * (1.0 + lax.erf(x * np.float32(0.7071067811865476)))


def _pe_const(T, D):
    pos = np.arange(T)[:, None].astype(np.float32)
    div = np.exp(np.arange(0, D, 2).astype(np.float32) * (-np.log(10000.0) / D))
    pe = np.zeros((T, D), dtype=np.float32)
    pe[:, 0::2] = np.sin(pos * div)
    pe[:, 1::2] = np.cos(pos * div)
    return jnp.asarray(pe)


def _router_route(x, pe, rW1, rb1, rW2, rb2, temp, NB):
    T, D = x.shape
    H = rW1.shape[1]
    E = rW2.shape[1]
    HB = 768
    nH = H // HB

    def body(x_ref, pe_ref, w1_ref, b1_ref, w2_ref, rb2_ref, temp_ref,
             h_ref, lg_ref, topv_ref, d0_ref, d1_ref, bexp_ref, loss_ref):
        j = pl.program_id(0)

        @pl.when(j == 0)
        def _():
            h_ref[...] = x_ref[...] + pe_ref[...]

        t = _gelu(jnp.dot(h_ref[...], w1_ref[...],
                          preferred_element_type=_F32) + b1_ref[...])
        contrib = jnp.dot(t, w2_ref[...], preferred_element_type=_F32)

        @pl.when(j == 0)
        def _():
            lg_ref[...] = contrib

        @pl.when(j != 0)
        def _():
            lg_ref[...] += contrib

        @pl.when(j == nH - 1)
        def _():
            _route_tail(lg_ref, rb2_ref, temp_ref, topv_ref, d0_ref,
                        d1_ref, bexp_ref, loss_ref)

    def _route_tail(lg_ref, rb2_ref, temp_ref, topv_ref, d0_ref, d1_ref,
                    bexp_ref, loss_ref):
        inv = 1.0 / (temp_ref[0, 0] + np.float32(1e-6))
        lg = (lg_ref[...] + rb2_ref[...]) * inv
        lg_ref[...] = lg
        m = jnp.max(lg, axis=-1, keepdims=True)
        ex = jnp.exp(lg - m)
        probs = ex / jnp.sum(ex, axis=-1, keepdims=True)
        iota8 = lax.broadcasted_iota(_I32, (T, E), 1)
        m1 = jnp.max(probs, axis=-1, keepdims=True)
        i1 = jnp.min(jnp.where(probs == m1, iota8, E), axis=-1, keepdims=True)
        A0 = iota8 == i1
        pr2 = jnp.where(A0, np.float32(-1.0), probs)
        m2 = jnp.max(pr2, axis=-1, keepdims=True)
        i2 = jnp.min(jnp.where(pr2 == m2, iota8, E), axis=-1, keepdims=True)
        A1 = iota8 == i2
        topv_ref[...] = jnp.concatenate([m1, m2], axis=-1)

        A0f = A0.astype(_F32)
        A1f = A1.astype(_F32)
        c0 = jnp.sum(A0f, axis=0, keepdims=True)
        c1 = jnp.sum(A1f, axis=0, keepdims=True)

        el = jnp.sum(probs, axis=0, keepdims=True)
        loss = jnp.mean(el * el) * np.float32(E)
        loss_ref[...] = loss[None, None]

        cnt = (c0 + c1).astype(_I32)
        nb_e = (cnt + (_BK - 1)) // _BK
        inc = nb_e
        sh = 1
        while sh < E:
            inc = inc + jnp.concatenate(
                [jnp.zeros((1, sh), _I32), inc[:, :-sh]], axis=-1)
            sh *= 2
        bstart = inc - nb_e
        slot_base = (bstart * _BK).astype(_F32)
        bid = lax.broadcasted_iota(_I32, (NB, 1), 0)
        bexp = jnp.sum((bid >= inc).astype(_I32), axis=-1, keepdims=True)
        used = bid < jnp.max(inc, axis=-1, keepdims=True)
        bexp_ref[...] = jnp.where(used, jnp.minimum(bexp, E - 1), E)

        def excl0(Af):
            c = Af
            s = 1
            while s < T:
                c = c + jnp.concatenate(
                    [jnp.zeros((s, E), _F32), c[:-s]], axis=0)
                s *= 2
            return c - Af

        R0 = excl0(A0f)
        R1 = excl0(A1f)
        d0 = jnp.sum(A0f * (slot_base + R0), axis=-1, keepdims=True)
        d1 = jnp.sum(A1f * (slot_base + c0 + R1), axis=-1, keepdims=True)
        d0_ref[...] = d0.astype(_I32)
        d1_ref[...] = d1.astype(_I32)

    full = lambda shape: pl.BlockSpec(shape, lambda j: tuple(0 for _ in shape))
    return pl.pallas_call(
        body,
        grid=(nH,),
        in_specs=[
            full((T, D)),
            full((T, D)),
            pl.BlockSpec((D, HB), lambda j: (0, j)),
            pl.BlockSpec((1, HB), lambda j: (0, j)),
            pl.BlockSpec((HB, E), lambda j: (j, 0)),
            full((1, E)),
            full((1, 1)),
        ],
        out_specs=[
            full((T, D)),
            full((T, E)),
            full((T, 2)),
            full((T, 1)),
            full((T, 1)),
            full((NB, 1)),
            full((1, 1)),
        ],
        out_shape=[
            jax.ShapeDtypeStruct((T, D), _F32),
            jax.ShapeDtypeStruct((T, E), _F32),
            jax.ShapeDtypeStruct((T, 2), _F32),
            jax.ShapeDtypeStruct((T, 1), _I32),
            jax.ShapeDtypeStruct((T, 1), _I32),
            jax.ShapeDtypeStruct((NB, 1), _I32),
            jax.ShapeDtypeStruct((1, 1), _F32),
        ],
    )(x, pe, rW1, rb1.reshape(1, H), rW2, rb2.reshape(1, E),
      temp.reshape(1, 1))


def _dispatch_sc(h, dest_all, P):
    T, D = h.shape
    NPAIR = dest_all.shape[0]
    info = plsc.get_sparse_core_info()
    NW = info.num_cores * info.num_subcores
    per_w = NPAIR // NW
    mesh = plsc.VectorSubcoreMesh(core_axis_name="c", subcore_axis_name="s")

    @functools.partial(
        pl.kernel,
        mesh=mesh,
        out_type=jax.ShapeDtypeStruct((P, D), _F32),
        scratch_types=[
            pltpu.VMEM((per_w,), _I32),
            pltpu.VMEM((per_w, D), _F32),
            pltpu.SemaphoreType.DMA,
        ],
    )
    def k(h_hbm, dest_hbm, xg_hbm, idx_v, rows_v, sem):
        wid = lax.axis_index("s") * info.num_cores + lax.axis_index("c")
        p0 = wid * per_w
        t0 = lax.rem(p0, T)
        pltpu.sync_copy(dest_hbm.at[pl.ds(p0, per_w)], idx_v)
        pltpu.sync_copy(h_hbm.at[pl.ds(t0, per_w)], rows_v)
        pltpu.async_copy(rows_v, xg_hbm.at[idx_v], sem).wait()

    return k(h, dest_all)


def _combine_sc(eo_pad, dest_all):
    P, D = eo_pad.shape
    NPAIR = dest_all.shape[0]
    info = plsc.get_sparse_core_info()
    NW = info.num_cores * info.num_subcores
    per_w = NPAIR // NW
    mesh = plsc.VectorSubcoreMesh(core_axis_name="c", subcore_axis_name="s")

    @functools.partial(
        pl.kernel,
        mesh=mesh,
        out_type=jax.ShapeDtypeStruct((NPAIR, D), _F32),
        scratch_types=[
            pltpu.VMEM((per_w,), _I32),
            pltpu.VMEM((per_w, D), _F32),
            pltpu.SemaphoreType.DMA,
        ],
    )
    def k(eo_hbm, dest_hbm, g_hbm, idx_v, rows_v, sem):
        wid = lax.axis_index("s") * info.num_cores + lax.axis_index("c")
        p0 = wid * per_w
        pltpu.sync_copy(dest_hbm.at[pl.ds(p0, per_w)], idx_v)
        pltpu.async_copy(eo_hbm.at[idx_v], rows_v, sem).wait()
        pltpu.sync_copy(rows_v, g_hbm.at[pl.ds(p0, per_w)])

    return k(eo_pad, dest_all)


def _experts(xg, bexp, eWg, ebg, eWa, eba, eWb, ebb, eg, eb):
    P, D = xg.shape
    E, SUB = ebg.shape
    NB = P // _BK
    DS = SUB * D
    wb_cat = eWb.reshape(E, DS, D)

    def body(bexp_ref, xg_ref, wg_ref, bg_ref, wa_ref, ba_ref, wb_ref,
             bb_ref, g_ref, b_ref, out_ref):
        b = pl.program_id(0)

        @pl.when(bexp_ref[b] < E)
        def _():
            xgb = xg_ref[...]
            gate_l = (jnp.dot(xgb, wg_ref[0], preferred_element_type=_F32)
                      + bg_ref[0])
            gm = jnp.max(gate_l, axis=-1, keepdims=True)
            ge = jnp.exp(gate_l - gm)
            gate = ge / jnp.sum(ge, axis=-1, keepdims=True)
            ag = jnp.concatenate(
                [gate[:, s:s + 1]
                 * _gelu(jnp.dot(xgb, wa_ref[0, s],
                                 preferred_element_type=_F32)
                         + ba_ref[0, s][None, :])
                 for s in range(SUB)], axis=1)
            mix = (jnp.dot(ag, wb_ref[0], preferred_element_type=_F32)
                   + jnp.dot(gate, bb_ref[0], preferred_element_type=_F32))
            r = xgb + mix
            mu = jnp.mean(r, axis=-1, keepdims=True)
            var = jnp.mean((r - mu) ** 2, axis=-1, keepdims=True)
            eo = (r - mu) * lax.rsqrt(var + np.float32(1e-5))
            out_ref[...] = eo * g_ref[0] + b_ref[0]

    def _emap(b, m):
        return (jnp.minimum(m[b], E - 1), 0, 0)

    grid_spec = pltpu.PrefetchScalarGridSpec(
        num_scalar_prefetch=1,
        grid=(NB,),
        in_specs=[
            pl.BlockSpec((_BK, D), lambda b, m: (b, 0)),
            pl.BlockSpec((1, D, SUB), _emap),
            pl.BlockSpec((1, 1, SUB), _emap),
            pl.BlockSpec((1, SUB, D, D),
                         lambda b, m: (jnp.minimum(m[b], E - 1), 0, 0, 0)),
            pl.BlockSpec((1, SUB, D), _emap),
            pl.BlockSpec((1, DS, D), _emap),
            pl.BlockSpec((1, SUB, D), _emap),
            pl.BlockSpec((1, 1, D), _emap),
            pl.BlockSpec((1, 1, D), _emap),
        ],
        out_specs=pl.BlockSpec((_BK, D), lambda b, m: (b, 0)),
    )
    return pl.pallas_call(
        body,
        grid_spec=grid_spec,
        out_shape=jax.ShapeDtypeStruct((P, D), _F32),
    )(bexp, xg, eWg, ebg.reshape(E, 1, SUB), eWa, eba, wb_cat, ebb,
      eg.reshape(E, 1, D), eb.reshape(E, 1, D))


def _final(g0, g1, topv, cW1, cb1, cW2, cb2, cg, cb):
    T, D = g0.shape
    D2 = cW1.shape[1]
    TB = 256
    nT = T // TB

    def body(g0_ref, g1_ref, tv_ref, w1_ref, b1_ref, w2_ref, b2_ref,
             g_ref, b_ref, out_ref):
        tv = tv_ref[...]
        comb = tv[:, 0:1] * g0_ref[...] + tv[:, 1:2] * g1_ref[...]
        y = _gelu(jnp.dot(comb, w1_ref[...], preferred_element_type=_F32)
                  + b1_ref[...])
        z = jnp.dot(y, w2_ref[...], preferred_element_type=_F32) + b2_ref[...]
        mu = jnp.mean(z, axis=-1, keepdims=True)
        var = jnp.mean((z - mu) ** 2, axis=-1, keepdims=True)
        out_ref[...] = ((z - mu) * lax.rsqrt(var + np.float32(1e-5))
                        * g_ref[...] + b_ref[...])

    return pl.pallas_call(
        body,
        grid=(nT,),
        in_specs=[
            pl.BlockSpec((TB, D), lambda i: (i, 0)),
            pl.BlockSpec((TB, D), lambda i: (i, 0)),
            pl.BlockSpec((TB, 2), lambda i: (i, 0)),
            pl.BlockSpec((D, D2), lambda i: (0, 0)),
            pl.BlockSpec((1, D2), lambda i: (0, 0)),
            pl.BlockSpec((D2, D), lambda i: (0, 0)),
            pl.BlockSpec((1, D), lambda i: (0, 0)),
            pl.BlockSpec((1, D), lambda i: (0, 0)),
            pl.BlockSpec((1, D), lambda i: (0, 0)),
        ],
        out_specs=pl.BlockSpec((TB, D), lambda i: (i, 0)),
        out_shape=jax.ShapeDtypeStruct((T, D), _F32),
    )(g0, g1, topv, cW1, cb1.reshape(1, D2), cW2, cb2.reshape(1, D),
      cg.reshape(1, D), cb.reshape(1, D))


def kernel(x, rW1, rb1, rW2, rb2, temp, eWg, ebg, eWa, eba, eWb, ebb,
           eg, eb, cW1, cb1, cW2, cb2, cg, cb):
    Bsz, T, D = x.shape
    E = rW2.shape[1]
    K = 2
    NPAIR = Bsz * T * K
    NB = -(-(NPAIR + E * (_BK - 1)) // _BK)
    P = NB * _BK

    xs = x.reshape(Bsz * T, D)
    pe = _pe_const(T, D)
    if Bsz > 1:
        pe = jnp.tile(pe, (Bsz, 1))

    h, logits, topv, d0, d1, bexp, loss = _router_route(
        xs, pe, rW1, rb1, rW2, rb2, temp, NB)
    dest_all = jnp.concatenate([d0.reshape(-1), d1.reshape(-1)])
    xg = _dispatch_sc(h, dest_all, P)
    eo_pad = _experts(xg, bexp.reshape(NB), eWg, ebg, eWa, eba, eWb, ebb,
                      eg, eb)
    g_all = _combine_sc(eo_pad, dest_all)
    out = _final(g_all[:Bsz * T], g_all[Bsz * T:], topv, cW1, cb1, cW2,
                 cb2, cg, cb)
    return (out.reshape(Bsz, T, D), logits.reshape(Bsz, T, E),
            loss.reshape(()))


# speedup vs baseline: 1.2620x; 1.0541x over previous
import functools

import numpy as np
import jax
import jax.numpy as jnp
from jax import lax
from jax.experimental import pallas as pl
from jax.experimental.pallas import tpu as pltpu
from jax.experimental.pallas import tpu_sc as plsc

_F32 = jnp.float32
_BF16 = jnp.bfloat16
_I32 = jnp.int32
_BK = 128


def _gelu(x):
    return 0.5 * x * (1.0 + lax.erf(x * np.float32(0.7071067811865476)))


def _pe_const(T, D):
    pos = np.arange(T)[:, None].astype(np.float32)
    div = np.exp(np.arange(0, D, 2).astype(np.float32) * (-np.log(10000.0) / D))
    pe = np.zeros((T, D), dtype=np.float32)
    pe[:, 0::2] = np.sin(pos * div)
    pe[:, 1::2] = np.cos(pos * div)
    return jnp.asarray(pe)


def _router_route(x, pe, rW1, rb1, rW2, rb2, temp, NB):
    T, D = x.shape
    H = rW1.shape[1]
    E = rW2.shape[1]
    HB = 768
    nH = H // HB

    def body(x_ref, pe_ref, w1_ref, b1_ref, w2_ref, rb2_ref, temp_ref,
             h_ref, lg_ref, topv_ref, dall_ref, bexp_ref, loss_ref):
        j = pl.program_id(0)

        @pl.when(j == 0)
        def _():
            h_ref[...] = x_ref[...] + pe_ref[...]

        t = _gelu(jnp.dot(h_ref[...], w1_ref[...],
                          preferred_element_type=_F32) + b1_ref[...])
        contrib = jnp.dot(t, w2_ref[...], preferred_element_type=_F32)

        @pl.when(j == 0)
        def _():
            lg_ref[...] = contrib

        @pl.when(j != 0)
        def _():
            lg_ref[...] += contrib

        @pl.when(j == nH - 1)
        def _():
            _route_tail(lg_ref, rb2_ref, temp_ref, topv_ref, dall_ref,
                        bexp_ref, loss_ref)

    def _route_tail(lg_ref, rb2_ref, temp_ref, topv_ref, dall_ref,
                    bexp_ref, loss_ref):
        inv = 1.0 / (temp_ref[0, 0] + np.float32(1e-6))
        lg = (lg_ref[...] + rb2_ref[...]) * inv
        lg_ref[...] = lg
        m = jnp.max(lg, axis=-1, keepdims=True)
        ex = jnp.exp(lg - m)
        probs = ex / jnp.sum(ex, axis=-1, keepdims=True)
        iota8 = lax.broadcasted_iota(_I32, (T, E), 1)
        m1 = jnp.max(probs, axis=-1, keepdims=True)
        i1 = jnp.min(jnp.where(probs == m1, iota8, E), axis=-1, keepdims=True)
        A0 = iota8 == i1
        pr2 = jnp.where(A0, np.float32(-1.0), probs)
        m2 = jnp.max(pr2, axis=-1, keepdims=True)
        i2 = jnp.min(jnp.where(pr2 == m2, iota8, E), axis=-1, keepdims=True)
        A1 = iota8 == i2
        topv_ref[...] = jnp.concatenate([m1, m2], axis=-1)

        A0f = A0.astype(_F32)
        A1f = A1.astype(_F32)
        c0 = jnp.sum(A0f, axis=0, keepdims=True)
        c1 = jnp.sum(A1f, axis=0, keepdims=True)

        el = jnp.sum(probs, axis=0, keepdims=True)
        loss = jnp.mean(el * el) * np.float32(E)
        loss_ref[...] = loss[None, None]

        cnt = (c0 + c1).astype(_I32)
        nb_e = (cnt + (_BK - 1)) // _BK
        inc = nb_e
        sh = 1
        while sh < E:
            inc = inc + jnp.concatenate(
                [jnp.zeros((1, sh), _I32), inc[:, :-sh]], axis=-1)
            sh *= 2
        bstart = inc - nb_e
        slot_base = (bstart * _BK).astype(_F32)
        bid = lax.broadcasted_iota(_I32, (NB, 1), 0)
        bexp = jnp.sum((bid >= inc).astype(_I32), axis=-1, keepdims=True)
        used = bid < jnp.max(inc, axis=-1, keepdims=True)
        bexp_ref[...] = jnp.where(used, jnp.minimum(bexp, E - 1), E)

        def excl0(Af):
            c = Af
            s = 1
            while s < T:
                c = c + jnp.concatenate(
                    [jnp.zeros((s, E), _F32), c[:-s]], axis=0)
                s *= 2
            return c - Af

        R0 = excl0(A0f)
        R1 = excl0(A1f)
        d0 = jnp.sum(A0f * (slot_base + R0), axis=-1, keepdims=True)
        d1 = jnp.sum(A1f * (slot_base + c0 + R1), axis=-1, keepdims=True)
        dall_ref[...] = jnp.concatenate([d0, d1], axis=0).astype(_I32)

    full = lambda shape: pl.BlockSpec(shape, lambda j: tuple(0 for _ in shape))
    return pl.pallas_call(
        body,
        grid=(nH,),
        in_specs=[
            full((T, D)),
            full((T, D)),
            pl.BlockSpec((D, HB), lambda j: (0, j)),
            pl.BlockSpec((1, HB), lambda j: (0, j)),
            pl.BlockSpec((HB, E), lambda j: (j, 0)),
            full((1, E)),
            full((1, 1)),
        ],
        out_specs=[
            full((T, D)),
            full((T, E)),
            full((T, 2)),
            full((2 * T, 1)),
            full((NB, 1)),
            full((1, 1)),
        ],
        out_shape=[
            jax.ShapeDtypeStruct((T, D), _F32),
            jax.ShapeDtypeStruct((T, E), _F32),
            jax.ShapeDtypeStruct((T, 2), _F32),
            jax.ShapeDtypeStruct((2 * T, 1), _I32),
            jax.ShapeDtypeStruct((NB, 1), _I32),
            jax.ShapeDtypeStruct((1, 1), _F32),
        ],
    )(x, pe, rW1, rb1.reshape(1, H), rW2, rb2.reshape(1, E),
      temp.reshape(1, 1))


def _dispatch_sc(h, dest_all, P):
    T, D = h.shape
    NPAIR = dest_all.shape[0]
    info = plsc.get_sparse_core_info()
    NW = info.num_cores * info.num_subcores
    per_w = NPAIR // NW
    mesh = plsc.VectorSubcoreMesh(core_axis_name="c", subcore_axis_name="s")

    @functools.partial(
        pl.kernel,
        mesh=mesh,
        out_type=jax.ShapeDtypeStruct((P, D), _F32),
        scratch_types=[
            pltpu.VMEM((per_w,), _I32),
            pltpu.VMEM((per_w, D), _F32),
            pltpu.SemaphoreType.DMA,
        ],
    )
    def k(h_hbm, dest_hbm, xg_hbm, idx_v, rows_v, sem):
        wid = lax.axis_index("s") * info.num_cores + lax.axis_index("c")
        p0 = wid * per_w
        t0 = lax.rem(p0, T)
        pltpu.sync_copy(dest_hbm.at[pl.ds(p0, per_w)], idx_v)
        pltpu.sync_copy(h_hbm.at[pl.ds(t0, per_w)], rows_v)
        pltpu.async_copy(rows_v, xg_hbm.at[idx_v], sem).wait()

    return k(h, dest_all)


def _combine_sc(eo_pad, dest_all):
    P, D = eo_pad.shape
    NPAIR = dest_all.shape[0]
    info = plsc.get_sparse_core_info()
    NW = info.num_cores * info.num_subcores
    per_w = NPAIR // NW
    mesh = plsc.VectorSubcoreMesh(core_axis_name="c", subcore_axis_name="s")

    @functools.partial(
        pl.kernel,
        mesh=mesh,
        out_type=jax.ShapeDtypeStruct((NPAIR, D), _F32),
        scratch_types=[
            pltpu.VMEM((per_w,), _I32),
            pltpu.VMEM((per_w, D), _F32),
            pltpu.SemaphoreType.DMA,
        ],
    )
    def k(eo_hbm, dest_hbm, g_hbm, idx_v, rows_v, sem):
        wid = lax.axis_index("s") * info.num_cores + lax.axis_index("c")
        p0 = wid * per_w
        pltpu.sync_copy(dest_hbm.at[pl.ds(p0, per_w)], idx_v)
        pltpu.async_copy(eo_hbm.at[idx_v], rows_v, sem).wait()
        pltpu.sync_copy(rows_v, g_hbm.at[pl.ds(p0, per_w)])

    return k(eo_pad, dest_all)


def _experts(xg, bexp, eWg, ebg, eWa, eba, eWb, ebb, eg, eb):
    P, D = xg.shape
    E, SUB = ebg.shape
    NB = P // _BK
    DS = SUB * D
    wb_cat = eWb.reshape(E, DS, D)

    def body(bexp_ref, xg_ref, wg_ref, bg_ref, wa_ref, ba_ref, wb_ref,
             bb_ref, g_ref, b_ref, out_ref):
        b = pl.program_id(0)

        @pl.when(bexp_ref[b] < E)
        def _():
            xgb = xg_ref[...]
            gate_l = (jnp.dot(xgb, wg_ref[0], preferred_element_type=_F32)
                      + bg_ref[0])
            gm = jnp.max(gate_l, axis=-1, keepdims=True)
            ge = jnp.exp(gate_l - gm)
            gate = ge / jnp.sum(ge, axis=-1, keepdims=True)
            ag = jnp.concatenate(
                [gate[:, s:s + 1]
                 * _gelu(jnp.dot(xgb, wa_ref[0, s],
                                 preferred_element_type=_F32)
                         + ba_ref[0, s][None, :])
                 for s in range(SUB)], axis=1)
            mix = (jnp.dot(ag, wb_ref[0], preferred_element_type=_F32)
                   + jnp.dot(gate, bb_ref[0], preferred_element_type=_F32))
            r = xgb + mix
            mu = jnp.mean(r, axis=-1, keepdims=True)
            var = jnp.mean((r - mu) ** 2, axis=-1, keepdims=True)
            eo = (r - mu) * lax.rsqrt(var + np.float32(1e-5))
            out_ref[...] = eo * g_ref[0] + b_ref[0]

    def _emap(b, m):
        return (jnp.minimum(m[b], E - 1), 0, 0)

    grid_spec = pltpu.PrefetchScalarGridSpec(
        num_scalar_prefetch=1,
        grid=(NB,),
        in_specs=[
            pl.BlockSpec((_BK, D), lambda b, m: (b, 0)),
            pl.BlockSpec((1, D, SUB), _emap),
            pl.BlockSpec((1, 1, SUB), _emap),
            pl.BlockSpec((1, SUB, D, D),
                         lambda b, m: (jnp.minimum(m[b], E - 1), 0, 0, 0)),
            pl.BlockSpec((1, SUB, D), _emap),
            pl.BlockSpec((1, DS, D), _emap),
            pl.BlockSpec((1, SUB, D), _emap),
            pl.BlockSpec((1, 1, D), _emap),
            pl.BlockSpec((1, 1, D), _emap),
        ],
        out_specs=pl.BlockSpec((_BK, D), lambda b, m: (b, 0)),
    )
    return pl.pallas_call(
        body,
        grid_spec=grid_spec,
        out_shape=jax.ShapeDtypeStruct((P, D), _F32),
    )(bexp, xg, eWg, ebg.reshape(E, 1, SUB), eWa, eba, wb_cat, ebb,
      eg.reshape(E, 1, D), eb.reshape(E, 1, D))


def _final(g_all, topv, cW1, cb1, cW2, cb2, cg, cb):
    T = g_all.shape[0] // 2
    D = g_all.shape[1]
    D2 = cW1.shape[1]
    TB = 256
    nT = T // TB

    def body(g0_ref, g1_ref, tv_ref, w1_ref, b1_ref, w2_ref, b2_ref,
             g_ref, b_ref, out_ref):
        tv = tv_ref[...]
        comb = tv[:, 0:1] * g0_ref[...] + tv[:, 1:2] * g1_ref[...]
        y = _gelu(jnp.dot(comb, w1_ref[...], preferred_element_type=_F32)
                  + b1_ref[...])
        z = jnp.dot(y, w2_ref[...], preferred_element_type=_F32) + b2_ref[...]
        mu = jnp.mean(z, axis=-1, keepdims=True)
        var = jnp.mean((z - mu) ** 2, axis=-1, keepdims=True)
        out_ref[...] = ((z - mu) * lax.rsqrt(var + np.float32(1e-5))
                        * g_ref[...] + b_ref[...])

    return pl.pallas_call(
        body,
        grid=(nT,),
        in_specs=[
            pl.BlockSpec((TB, D), lambda i: (i, 0)),
            pl.BlockSpec((TB, D), lambda i: (i + nT, 0)),
            pl.BlockSpec((TB, 2), lambda i: (i, 0)),
            pl.BlockSpec((D, D2), lambda i: (0, 0)),
            pl.BlockSpec((1, D2), lambda i: (0, 0)),
            pl.BlockSpec((D2, D), lambda i: (0, 0)),
            pl.BlockSpec((1, D), lambda i: (0, 0)),
            pl.BlockSpec((1, D), lambda i: (0, 0)),
            pl.BlockSpec((1, D), lambda i: (0, 0)),
        ],
        out_specs=pl.BlockSpec((TB, D), lambda i: (i, 0)),
        out_shape=jax.ShapeDtypeStruct((T, D), _F32),
    )(g_all, g_all, topv, cW1, cb1.reshape(1, D2), cW2, cb2.reshape(1, D),
      cg.reshape(1, D), cb.reshape(1, D))


def kernel(x, rW1, rb1, rW2, rb2, temp, eWg, ebg, eWa, eba, eWb, ebb,
           eg, eb, cW1, cb1, cW2, cb2, cg, cb):
    Bsz, T, D = x.shape
    E = rW2.shape[1]
    K = 2
    NPAIR = Bsz * T * K
    NB = -(-(NPAIR + E * (_BK - 1)) // _BK)
    P = NB * _BK

    xs = x.reshape(Bsz * T, D)
    pe = _pe_const(T, D)
    if Bsz > 1:
        pe = jnp.tile(pe, (Bsz, 1))

    h, logits, topv, dall, bexp, loss = _router_route(
        xs, pe, rW1, rb1, rW2, rb2, temp, NB)
    dest_all = dall.reshape(-1)
    xg = _dispatch_sc(h, dest_all, P)
    eo_pad = _experts(xg, bexp.reshape(NB), eWg, ebg, eWa, eba, eWb, ebb,
                      eg, eb)
    g_all = _combine_sc(eo_pad, dest_all)
    out = _final(g_all, topv, cW1, cb1, cW2, cb2, cg, cb)
    return (out.reshape(Bsz, T, D), logits.reshape(Bsz, T, E),
            loss.reshape(()))
